# Initial kernel scaffold; baseline (speedup 1.0000x reference)
import jax
import jax.numpy as jnp
from jax.experimental import pallas as pl


def kernel(h_clean, phones, target_gender, prototypes, proto_phones, proto_genders):
    raise NotImplementedError("write your pallas kernel here")


# speedup vs baseline: 2.6909x; 2.6909x over previous
import functools

import jax
import jax.numpy as jnp
from jax import lax
from jax.experimental import pallas as pl
from jax.experimental.pallas import tpu as pltpu
from jax.experimental.pallas import tpu_sc as plsc

K = 8
BIG = 1e9
T_TILE = 256
N_BLK = 512
LANES = 128


def _topk_body(q2_ref, ph_ref, p2_ref, code_ref, h_ref, pr_ref,
               idx_out_ref, w_out_ref, vals, inds, *, n_blocks):
    j = pl.program_id(1)

    @pl.when(j == 0)
    def _init():
        vals[...] = jnp.full((K, T_TILE, LANES), jnp.inf, jnp.float32)
        inds[...] = jnp.zeros((K, T_TILE, LANES), jnp.int32)

    hb = h_ref[...]
    pb = pr_ref[...]
    cross = lax.dot_general(
        hb.astype(jnp.bfloat16), pb.astype(jnp.bfloat16),
        dimension_numbers=(((1,), (1,)), ((), ())),
        preferred_element_type=jnp.float32,
    )
    d2 = (q2_ref[...] + p2_ref[...]) - 2.0 * cross
    dist = jnp.sqrt(jnp.maximum(d2, 1e-12))
    maskb = code_ref[...] == ph_ref[...]
    md = jnp.where(maskb, dist, jnp.float32(BIG))

    for g in range(N_BLK // LANES):
        v = md[:, g * LANES:(g + 1) * LANES]
        vidx = (j * N_BLK + g * LANES
                + lax.broadcasted_iota(jnp.int32, (T_TILE, LANES), 1))
        olds = [vals[s] for s in range(K)]
        oldi = [inds[s] for s in range(K)]
        c = [v < olds[s] for s in range(K)]
        for s in range(K):
            if s == 0:
                vals[0] = jnp.where(c[0], v, olds[0])
                inds[0] = jnp.where(c[0], vidx, oldi[0])
            else:
                vals[s] = jnp.where(c[s], jnp.where(c[s - 1], olds[s - 1], v),
                                    olds[s])
                inds[s] = jnp.where(c[s], jnp.where(c[s - 1], oldi[s - 1], vidx),
                                    oldi[s])

    @pl.when(j == n_blocks - 1)
    def _merge():
        cv = jnp.stack([vals[s] for s in range(K)])
        ci = jnp.stack([inds[s] for s in range(K)])
        outd, outi = [], []
        for _ in range(K):
            m = jnp.min(jnp.min(cv, axis=0), axis=1, keepdims=True)
            eq = cv == m[None, :, :]
            imin = jnp.min(jnp.min(jnp.where(eq, ci, jnp.int32(2**30)), axis=0),
                           axis=1, keepdims=True)
            outd.append(m)
            outi.append(imin)
            kill = eq & (ci == imin[None, :, :])
            cv = jnp.where(kill, jnp.inf, cv)
        topd = jnp.concatenate(outd, axis=1)
        topi = jnp.concatenate(outi, axis=1)
        unnorm = jnp.exp(-(topd - topd[:, 0:1]))
        w = unnorm / jnp.sum(unnorm, axis=1, keepdims=True)
        idx_out_ref[...] = topi
        w_out_ref[...] = w


def _make_sc_gather(V, D, B):
    nw = 32
    b_per_w = B // nw
    mesh = plsc.VectorSubcoreMesh(core_axis_name="c", subcore_axis_name="s")

    @functools.partial(
        pl.kernel, mesh=mesh,
        out_type=jax.ShapeDtypeStruct((B, D), jnp.float32),
        scratch_types=[
            pltpu.VMEM((b_per_w,), jnp.int32),
            pltpu.VMEM((b_per_w, D), jnp.float32),
            pltpu.SemaphoreType.DMA,
        ],
    )
    def sc_gather(table_hbm, idx_hbm, out_hbm, idx_v, rows_v, sem):
        wid = lax.axis_index("s") * 2 + lax.axis_index("c")
        base = wid * b_per_w
        pltpu.sync_copy(idx_hbm.at[pl.ds(base, b_per_w)], idx_v)
        pltpu.async_copy(table_hbm.at[idx_v], rows_v, sem).wait()
        pltpu.sync_copy(rows_v, out_hbm.at[pl.ds(base, b_per_w)])

    return sc_gather


def _combine_body(g_ref, w_ref, out_ref):
    g = g_ref[...]
    w = w_ref[...]
    out_ref[...] = jnp.sum(w[:, :, None] * g, axis=1)


def _run_combine(gathered, w):
    T = w.shape[0]
    g3 = gathered.reshape(T, K, 256)
    return pl.pallas_call(
        _combine_body,
        grid=(T // T_TILE,),
        in_specs=[
            pl.BlockSpec((T_TILE, K, 256), lambda i: (i, 0, 0)),
            pl.BlockSpec((T_TILE, K), lambda i: (i, 0)),
        ],
        out_specs=pl.BlockSpec((T_TILE, 256), lambda i: (i, 0)),
        out_shape=jax.ShapeDtypeStruct((T, 256), jnp.float32),
    )(g3, w)


def kernel(h_clean, phones, target_gender, prototypes, proto_phones,
           proto_genders):
    T, D = h_clean.shape
    N = prototypes.shape[0]
    q2 = jnp.sum(h_clean * h_clean, axis=1, keepdims=True)
    p2 = jnp.sum(prototypes * prototypes, axis=1)[None, :]
    code = jnp.where(proto_genders == target_gender, proto_phones,
                     jnp.int32(-1)).astype(jnp.int32)
    code2d = code.reshape(1, N)
    phones2d = phones.astype(jnp.int32).reshape(T, 1)

    n_t, n_blocks = T // T_TILE, N // N_BLK
    kern = functools.partial(_topk_body, n_blocks=n_blocks)
    topi, w = pl.pallas_call(
        kern,
        grid=(n_t, n_blocks),
        in_specs=[
            pl.BlockSpec((T_TILE, 1), lambda i, j: (i, 0)),
            pl.BlockSpec((T_TILE, 1), lambda i, j: (i, 0)),
            pl.BlockSpec((1, N_BLK), lambda i, j: (0, j)),
            pl.BlockSpec((1, N_BLK), lambda i, j: (0, j)),
            pl.BlockSpec((T_TILE, D), lambda i, j: (i, 0)),
            pl.BlockSpec((N_BLK, D), lambda i, j: (j, 0)),
        ],
        out_specs=[
            pl.BlockSpec((T_TILE, K), lambda i, j: (i, 0)),
            pl.BlockSpec((T_TILE, K), lambda i, j: (i, 0)),
        ],
        out_shape=[
            jax.ShapeDtypeStruct((T, K), jnp.int32),
            jax.ShapeDtypeStruct((T, K), jnp.float32),
        ],
        scratch_shapes=[
            pltpu.VMEM((K, T_TILE, LANES), jnp.float32),
            pltpu.VMEM((K, T_TILE, LANES), jnp.int32),
        ],
        compiler_params=pltpu.CompilerParams(
            dimension_semantics=("arbitrary", "arbitrary"),
        ),
    )(q2, phones2d, p2, code2d, h_clean, prototypes)

    idx_flat = topi.reshape(T * K)
    gathered = _make_sc_gather(N, D, T * K)(prototypes, idx_flat)
    return _run_combine(gathered, w)


# speedup vs baseline: 2.7256x; 1.0129x over previous
import functools

import jax
import jax.numpy as jnp
from jax import lax
from jax.experimental import pallas as pl
from jax.experimental.pallas import tpu as pltpu
from jax.experimental.pallas import tpu_sc as plsc

K = 8
BIG = 1e9
T_TILE = 256
N_BLK = 512
LANES = 128


def _topk_body(q2_ref, ph_ref, p2_ref, code_ref, h_ref, pr_ref,
               idx_out_ref, w_out_ref, vals, inds, *, n_blocks):
    j = pl.program_id(1)

    @pl.when(j == 0)
    def _init():
        vals[...] = jnp.full((K, T_TILE, LANES), jnp.inf, jnp.float32)
        inds[...] = jnp.zeros((K, T_TILE, LANES), jnp.int32)

    hb = h_ref[...]
    pb = pr_ref[...]
    cross = lax.dot_general(
        hb.astype(jnp.bfloat16), pb.astype(jnp.bfloat16),
        dimension_numbers=(((1,), (1,)), ((), ())),
        preferred_element_type=jnp.float32,
    )
    d2 = (q2_ref[...] + p2_ref[...]) - 2.0 * cross
    dist = jnp.sqrt(jnp.maximum(d2, 1e-12))
    maskb = code_ref[...] == ph_ref[...]
    md = jnp.where(maskb, dist, jnp.float32(BIG))

    cur_v = [vals[s] for s in range(K)]
    cur_i = [inds[s] for s in range(K)]
    for g in range(N_BLK // LANES):
        v = md[:, g * LANES:(g + 1) * LANES]
        vidx = (j * N_BLK + g * LANES
                + lax.broadcasted_iota(jnp.int32, (T_TILE, LANES), 1))
        c = [v < cur_v[s] for s in range(K)]
        new_v, new_i = [], []
        for s in range(K):
            if s == 0:
                new_v.append(jnp.where(c[0], v, cur_v[0]))
                new_i.append(jnp.where(c[0], vidx, cur_i[0]))
            else:
                new_v.append(jnp.where(c[s], jnp.where(c[s - 1], cur_v[s - 1],
                                                       v), cur_v[s]))
                new_i.append(jnp.where(c[s], jnp.where(c[s - 1], cur_i[s - 1],
                                                       vidx), cur_i[s]))
        cur_v, cur_i = new_v, new_i
    for s in range(K):
        vals[s] = cur_v[s]
        inds[s] = cur_i[s]

    @pl.when(j == n_blocks - 1)
    def _merge():
        cv = jnp.stack([vals[s] for s in range(K)])
        ci = jnp.stack([inds[s] for s in range(K)])
        outd, outi = [], []
        for _ in range(K):
            m = jnp.min(jnp.min(cv, axis=0), axis=1, keepdims=True)
            eq = cv == m[None, :, :]
            imin = jnp.min(jnp.min(jnp.where(eq, ci, jnp.int32(2**30)), axis=0),
                           axis=1, keepdims=True)
            outd.append(m)
            outi.append(imin)
            kill = eq & (ci == imin[None, :, :])
            cv = jnp.where(kill, jnp.inf, cv)
        topd = jnp.concatenate(outd, axis=1)
        topi = jnp.concatenate(outi, axis=1)
        unnorm = jnp.exp(-(topd - topd[:, 0:1]))
        w = unnorm / jnp.sum(unnorm, axis=1, keepdims=True)
        idx_out_ref[...] = topi
        w_out_ref[...] = w


def _make_sc_gather(V, D, B):
    nw = 32
    b_per_w = B // nw
    mesh = plsc.VectorSubcoreMesh(core_axis_name="c", subcore_axis_name="s")

    @functools.partial(
        pl.kernel, mesh=mesh,
        out_type=jax.ShapeDtypeStruct((B, D), jnp.float32),
        scratch_types=[
            pltpu.VMEM((b_per_w,), jnp.int32),
            pltpu.VMEM((b_per_w, D), jnp.float32),
            pltpu.SemaphoreType.DMA,
        ],
    )
    def sc_gather(table_hbm, idx_hbm, out_hbm, idx_v, rows_v, sem):
        wid = lax.axis_index("s") * 2 + lax.axis_index("c")
        base = wid * b_per_w
        pltpu.sync_copy(idx_hbm.at[pl.ds(base, b_per_w)], idx_v)
        pltpu.async_copy(table_hbm.at[idx_v], rows_v, sem).wait()
        pltpu.sync_copy(rows_v, out_hbm.at[pl.ds(base, b_per_w)])

    return sc_gather


def _combine_body(g_ref, w_ref, out_ref):
    g = g_ref[...]
    w = w_ref[...]
    out_ref[...] = jnp.sum(w[:, :, None] * g, axis=1)


def _run_combine(gathered, w):
    T = w.shape[0]
    g3 = gathered.reshape(T, K, 256)
    return pl.pallas_call(
        _combine_body,
        grid=(T // T_TILE,),
        in_specs=[
            pl.BlockSpec((T_TILE, K, 256), lambda i: (i, 0, 0)),
            pl.BlockSpec((T_TILE, K), lambda i: (i, 0)),
        ],
        out_specs=pl.BlockSpec((T_TILE, 256), lambda i: (i, 0)),
        out_shape=jax.ShapeDtypeStruct((T, 256), jnp.float32),
    )(g3, w)


def kernel(h_clean, phones, target_gender, prototypes, proto_phones,
           proto_genders):
    T, D = h_clean.shape
    N = prototypes.shape[0]
    q2 = jnp.sum(h_clean * h_clean, axis=1, keepdims=True)
    p2 = jnp.sum(prototypes * prototypes, axis=1)[None, :]
    code = jnp.where(proto_genders == target_gender, proto_phones,
                     jnp.int32(-1)).astype(jnp.int32)
    code2d = code.reshape(1, N)
    phones2d = phones.astype(jnp.int32).reshape(T, 1)

    n_t, n_blocks = T // T_TILE, N // N_BLK
    kern = functools.partial(_topk_body, n_blocks=n_blocks)
    topi, w = pl.pallas_call(
        kern,
        grid=(n_t, n_blocks),
        in_specs=[
            pl.BlockSpec((T_TILE, 1), lambda i, j: (i, 0)),
            pl.BlockSpec((T_TILE, 1), lambda i, j: (i, 0)),
            pl.BlockSpec((1, N_BLK), lambda i, j: (0, j)),
            pl.BlockSpec((1, N_BLK), lambda i, j: (0, j)),
            pl.BlockSpec((T_TILE, D), lambda i, j: (i, 0)),
            pl.BlockSpec((N_BLK, D), lambda i, j: (j, 0)),
        ],
        out_specs=[
            pl.BlockSpec((T_TILE, K), lambda i, j: (i, 0)),
            pl.BlockSpec((T_TILE, K), lambda i, j: (i, 0)),
        ],
        out_shape=[
            jax.ShapeDtypeStruct((T, K), jnp.int32),
            jax.ShapeDtypeStruct((T, K), jnp.float32),
        ],
        scratch_shapes=[
            pltpu.VMEM((K, T_TILE, LANES), jnp.float32),
            pltpu.VMEM((K, T_TILE, LANES), jnp.int32),
        ],
        compiler_params=pltpu.CompilerParams(
            dimension_semantics=("arbitrary", "arbitrary"),
        ),
    )(q2, phones2d, p2, code2d, h_clean, prototypes)

    idx_flat = topi.reshape(T * K)
    gathered = _make_sc_gather(N, D, T * K)(prototypes, idx_flat)
    return _run_combine(gathered, w)
